# Initial kernel scaffold; baseline (speedup 1.0000x reference)
import jax
import jax.numpy as jnp
from jax.experimental import pallas as pl


def kernel(x, edge_index, batch, W0, b0, W1, b1, W2, b2, conv_b, gru_Wih, gru_Whh, gru_bih, gru_bhh, lstm_Wih, lstm_Whh, lstm_bih, lstm_bhh):
    raise NotImplementedError("write your pallas kernel here")


# speedup vs baseline: 12.7592x; 12.7592x over previous
import functools

import jax
import jax.numpy as jnp
from jax import lax
from jax.experimental import pallas as pl
from jax.experimental.pallas import tpu as pltpu
from jax.experimental.pallas import tpu_sc as plsc

N, E, F, D, B = 10000, 320000, 128, 16, 64

NC, NS = 2, 16
NW = NC * NS

CHUNK = 128
NCHUNK = -(-E // (NW * CHUNK))
PER_W = NCHUNK * CHUNK
EPAD = NW * PER_W
RPT = -(-N // NS) // 8 * 8 + 8
NPAD = RPT * NS


def _sc_seg_sum_body(with_counts, table, srcp, dstp, *refs):
    if with_counts:
        out_acc, out_cnt, sidx_v, didx_v, rows_v, ones_v, zbuf, acc_sh, cnt_sh, sem = refs
    else:
        out_acc, sidx_v, didx_v, rows_v, zbuf, acc_sh, sem = refs
        out_cnt = ones_v = cnt_sh = None

    c = lax.axis_index("c")
    s = lax.axis_index("s")
    w = s * NC + c

    def _zero_row(i, _):
        zbuf[i, :] = jnp.zeros((D,), jnp.float32)
        return 0

    lax.fori_loop(0, RPT, _zero_row, 0)
    pltpu.sync_copy(zbuf, acc_sh.at[pl.ds(s * RPT, RPT)])
    if with_counts:
        def _one_row(i, _):
            ones_v[i, :] = jnp.ones((D,), jnp.float32)
            return 0

        lax.fori_loop(0, CHUNK, _one_row, 0)
        pltpu.sync_copy(zbuf, cnt_sh.at[pl.ds(s * RPT, RPT)])
    plsc.subcore_barrier()

    pltpu.sync_copy(srcp.at[pl.ds(w * PER_W, PER_W)], sidx_v)

    def _chunk(j, _):
        base = w * PER_W + j * CHUNK
        pltpu.sync_copy(dstp.at[pl.ds(base, CHUNK)], didx_v)
        pltpu.async_copy(
            table.at[sidx_v.at[pl.ds(j * CHUNK, CHUNK)]], rows_v, sem
        ).wait()
        pltpu.sync_copy(rows_v, acc_sh.at[didx_v], add=True)
        if with_counts:
            pltpu.sync_copy(ones_v, cnt_sh.at[didx_v], add=True)
        return 0

    lax.fori_loop(0, NCHUNK, _chunk, 0)
    plsc.subcore_barrier()

    pltpu.sync_copy(
        acc_sh.at[pl.ds(s * RPT, RPT)], out_acc.at[c, pl.ds(s * RPT, RPT)]
    )
    if with_counts:
        pltpu.sync_copy(
            cnt_sh.at[pl.ds(s * RPT, RPT)], out_cnt.at[c, pl.ds(s * RPT, RPT)]
        )


@functools.lru_cache(maxsize=None)
def _make_sc_seg_sum(with_counts):
    mesh = plsc.VectorSubcoreMesh(
        core_axis_name="c", subcore_axis_name="s", num_cores=NC, num_subcores=NS
    )
    out_type = [jax.ShapeDtypeStruct((NC, NPAD, D), jnp.float32)]
    scratch = [
        pltpu.VMEM((PER_W,), jnp.int32),
        pltpu.VMEM((CHUNK,), jnp.int32),
        pltpu.VMEM((CHUNK, D), jnp.float32),
        pltpu.VMEM((RPT, D), jnp.float32),
        pltpu.VMEM_SHARED((NPAD, D), jnp.float32),
        pltpu.SemaphoreType.DMA,
    ]
    if with_counts:
        out_type.append(jax.ShapeDtypeStruct((NC, NPAD, D), jnp.float32))
        scratch.insert(3, pltpu.VMEM((CHUNK, D), jnp.float32))
        scratch.insert(6, pltpu.VMEM_SHARED((NPAD, D), jnp.float32))
    return pl.kernel(
        functools.partial(_sc_seg_sum_body, with_counts),
        out_type=out_type,
        mesh=mesh,
        scratch_types=scratch,
        compiler_params=pltpu.CompilerParams(use_tc_tiling_on_sc=False),
    )


def _t0_body(x_ref, w0_ref, b0_ref, w1_ref, b1_ref, w2_ref, b2_ref,
             out_ref, wf_ref):
    out_ref[...] = jax.nn.relu(
        jnp.dot(x_ref[...], w0_ref[...], preferred_element_type=jnp.float32)
        + b0_ref[...]
    )
    hidden = jax.nn.relu(w1_ref[...] + b1_ref[...])
    wf_ref[...] = (
        jnp.dot(hidden, w2_ref[...], preferred_element_type=jnp.float32)
        + b2_ref[...]
    )


_t0_call = pl.pallas_call(
    _t0_body,
    out_shape=[
        jax.ShapeDtypeStruct((N, D), jnp.float32),
        jax.ShapeDtypeStruct((1, D * D), jnp.float32),
    ],
)


def _t1_body(acc_ref, cnt_ref, h_ref, wf_ref, cb_ref,
             wih_ref, whh_ref, bih_ref, bhh_ref, out_ref):
    seg = acc_ref[0, :N, :] + acc_ref[1, :N, :]
    cnt = cnt_ref[0, :N, :] + cnt_ref[1, :N, :]
    mean = seg / jnp.maximum(cnt, 1.0)
    m = jax.nn.relu(
        jnp.dot(mean, wf_ref[...], preferred_element_type=jnp.float32)
        + cb_ref[...]
    )
    h = h_ref[...]
    gi = jnp.dot(m, wih_ref[...], preferred_element_type=jnp.float32) + bih_ref[...]
    gh = jnp.dot(h, whh_ref[...], preferred_element_type=jnp.float32) + bhh_ref[...]
    r = jax.nn.sigmoid(gi[:, :D] + gh[:, :D])
    z = jax.nn.sigmoid(gi[:, D:2 * D] + gh[:, D:2 * D])
    n = jnp.tanh(gi[:, 2 * D:] + r * gh[:, 2 * D:])
    out_ref[...] = (1.0 - z) * n + z * h


_t1_call = pl.pallas_call(
    _t1_body,
    out_shape=jax.ShapeDtypeStruct((N, D), jnp.float32),
)


def _t2_body(out_node_ref, batch_ref, wih_ref, whh_ref, bih_ref, bhh_ref,
             q_ref):
    out = out_node_ref[...]
    bidx = batch_ref[...]
    cols = lax.broadcasted_iota(jnp.int32, (N, B), 1)
    oh = (bidx == cols).astype(jnp.float32)
    q_star = jnp.zeros((B, 2 * D), jnp.float32)
    hs = jnp.zeros((B, D), jnp.float32)
    cs = jnp.zeros((B, D), jnp.float32)
    for _ in range(3):
        gates = (
            jnp.dot(q_star, wih_ref[...], preferred_element_type=jnp.float32)
            + bih_ref[...]
            + jnp.dot(hs, whh_ref[...], preferred_element_type=jnp.float32)
            + bhh_ref[...]
        )
        ig = jax.nn.sigmoid(gates[:, :D])
        fg = jax.nn.sigmoid(gates[:, D:2 * D])
        gg = jnp.tanh(gates[:, 2 * D:3 * D])
        og = jax.nn.sigmoid(gates[:, 3 * D:])
        cs = fg * cs + ig * gg
        hs = og * jnp.tanh(cs)
        qb = jnp.dot(oh, hs, preferred_element_type=jnp.float32)
        e = jnp.sum(out * qb, axis=1, keepdims=True)
        e_masked = jnp.where(oh > 0.0, e, -1e30)
        e_max = jnp.max(e_masked, axis=0, keepdims=True)
        e_max_n = jnp.dot(oh, e_max.T, preferred_element_type=jnp.float32)
        a_un = jnp.exp(e - e_max_n)
        denom = lax.dot_general(
            oh, a_un, (((0,), (0,)), ((), ())),
            preferred_element_type=jnp.float32,
        )
        den_n = jnp.dot(oh, denom, preferred_element_type=jnp.float32)
        a = a_un / (den_n + 1e-16)
        r = lax.dot_general(
            oh, a * out, (((0,), (0,)), ((), ())),
            preferred_element_type=jnp.float32,
        )
        q_star = jnp.concatenate([hs, r], axis=1)
    q_ref[...] = q_star


_t2_call = pl.pallas_call(
    _t2_body,
    out_shape=jax.ShapeDtypeStruct((B, 2 * D), jnp.float32),
)


def kernel(x, edge_index, batch, W0, b0, W1, b1, W2, b2, conv_b,
           gru_Wih, gru_Whh, gru_bih, gru_bhh,
           lstm_Wih, lstm_Whh, lstm_bih, lstm_bhh):
    src = edge_index[0]
    dst = edge_index[1]
    npad = EPAD - E
    src_p = jnp.concatenate([src, jnp.zeros((npad,), jnp.int32)])
    dst_p = jnp.concatenate([dst, jnp.full((npad,), NPAD - 1, jnp.int32)])

    out0, wf_flat = _t0_call(
        x, W0, b0.reshape(1, D), W1, b1.reshape(1, D), W2, b2.reshape(1, D * D)
    )
    wfix = wf_flat.reshape(D, D)
    cb = conv_b.reshape(1, D)
    wihT, bihT = gru_Wih.T, gru_bih.reshape(1, 3 * D)
    whhT, bhhT = gru_Whh.T, gru_bhh.reshape(1, 3 * D)

    h = out0
    cnt2 = None
    for layer in range(3):
        if layer == 0:
            acc2, cnt2 = _make_sc_seg_sum(True)(h, src_p, dst_p)
        else:
            (acc2,) = _make_sc_seg_sum(False)(h, src_p, dst_p)
        h = _t1_call(acc2, cnt2, h, wfix, cb, wihT, whhT, bihT, bhhT)

    q_star = _t2_call(
        h, batch.reshape(N, 1), lstm_Wih.T, lstm_Whh.T,
        lstm_bih.reshape(1, 4 * D), lstm_bhh.reshape(1, 4 * D),
    )
    return (q_star, h)


# speedup vs baseline: 19.4126x; 1.5215x over previous
import functools

import jax
import jax.numpy as jnp
from jax import lax
from jax.experimental import pallas as pl
from jax.experimental.pallas import tpu as pltpu
from jax.experimental.pallas import tpu_sc as plsc

N, E, F, D, B = 10000, 320000, 128, 16, 64

NC, NS = 2, 16
NW = NC * NS

CHUNK = 128
K = 4
NB = 2 * K
NCHUNK = -(-E // (NW * CHUNK * NB)) * NB
PER_W = NCHUNK * CHUNK
EPAD = NW * PER_W
NBLK = NCHUNK // NB
RPT = -(-N // NS) // 8 * 8 + 8
NPAD = RPT * NS


def _sc_seg_sum_body(with_counts, table, srcp, dstp, *refs):
    if with_counts:
        (out_acc, out_cnt, sidx2, didx2, rows, ones_v, zbuf,
         acc_sh, cnt_sh, gsem, ssem, csem) = refs
    else:
        out_acc, sidx2, didx2, rows, zbuf, acc_sh, gsem, ssem = refs
        out_cnt = ones_v = cnt_sh = csem = None

    c = lax.axis_index("c")
    s = lax.axis_index("s")
    w = s * NC + c

    def _zero_row(i, _):
        zbuf[i, :] = jnp.zeros((D,), jnp.float32)
        return 0

    lax.fori_loop(0, RPT, _zero_row, 0)
    pltpu.sync_copy(zbuf, acc_sh.at[pl.ds(s * RPT, RPT)])
    if with_counts:
        def _one_row(i, _):
            ones_v[i, :] = jnp.ones((D,), jnp.float32)
            return 0

        lax.fori_loop(0, CHUNK, _one_row, 0)
        pltpu.sync_copy(zbuf, cnt_sh.at[pl.ds(s * RPT, RPT)])

    pltpu.sync_copy(srcp.at[w], sidx2)
    pltpu.sync_copy(dstp.at[w], didx2)
    plsc.subcore_barrier()

    def g_issue(j, b):
        pltpu.async_copy(table.at[sidx2.at[j]], rows.at[b], gsem.at[b])

    def g_wait(j, b):
        pltpu.make_async_copy(
            table.at[sidx2.at[j]], rows.at[b], gsem.at[b]
        ).wait()

    def s_issue(j, b):
        pltpu.async_copy(rows.at[b], acc_sh.at[didx2.at[j]], ssem.at[b],
                         add=True)
        if with_counts:
            pltpu.async_copy(ones_v, cnt_sh.at[didx2.at[j]], csem.at[b],
                             add=True)

    def s_wait(j, b):
        pltpu.make_async_copy(
            rows.at[b], acc_sh.at[didx2.at[j]], ssem.at[b]
        ).wait()
        if with_counts:
            pltpu.make_async_copy(
                ones_v, cnt_sh.at[didx2.at[j]], csem.at[b]
            ).wait()

    for b in range(NB):
        g_issue(b, b)

    def _blk(t, _):
        j0 = t * NB
        for b in range(NB):
            j = j0 + b
            g_wait(j, b)
            s_issue(j, b)
            br = (b - K) % NB
            jr = j0 + br + (NB if b >= K else 0)
            @pl.when(jnp.logical_and(jr >= NB, jr < NCHUNK))
            def _():
                s_wait(jr - NB, br)
                g_issue(jr, br)
        return 0

    lax.fori_loop(0, NBLK, _blk, 0)
    for b in range(NB):
        s_wait((NBLK - 1) * NB + b, b)
    plsc.subcore_barrier()

    pltpu.sync_copy(
        acc_sh.at[pl.ds(s * RPT, RPT)], out_acc.at[c, pl.ds(s * RPT, RPT)]
    )
    if with_counts:
        pltpu.sync_copy(
            cnt_sh.at[pl.ds(s * RPT, RPT)], out_cnt.at[c, pl.ds(s * RPT, RPT)]
        )


@functools.lru_cache(maxsize=None)
def _make_sc_seg_sum(with_counts):
    mesh = plsc.VectorSubcoreMesh(
        core_axis_name="c", subcore_axis_name="s", num_cores=NC, num_subcores=NS
    )
    out_type = [jax.ShapeDtypeStruct((NC, NPAD, D), jnp.float32)]
    if with_counts:
        out_type.append(jax.ShapeDtypeStruct((NC, NPAD, D), jnp.float32))
        scratch = [
            pltpu.VMEM((NCHUNK, CHUNK), jnp.int32),
            pltpu.VMEM((NCHUNK, CHUNK), jnp.int32),
            pltpu.VMEM((NB, CHUNK, D), jnp.float32),
            pltpu.VMEM((CHUNK, D), jnp.float32),
            pltpu.VMEM((RPT, D), jnp.float32),
            pltpu.VMEM_SHARED((NPAD, D), jnp.float32),
            pltpu.VMEM_SHARED((NPAD, D), jnp.float32),
            pltpu.SemaphoreType.DMA((NB,)),
            pltpu.SemaphoreType.DMA((NB,)),
            pltpu.SemaphoreType.DMA((NB,)),
        ]
    else:
        scratch = [
            pltpu.VMEM((NCHUNK, CHUNK), jnp.int32),
            pltpu.VMEM((NCHUNK, CHUNK), jnp.int32),
            pltpu.VMEM((NB, CHUNK, D), jnp.float32),
            pltpu.VMEM((RPT, D), jnp.float32),
            pltpu.VMEM_SHARED((NPAD, D), jnp.float32),
            pltpu.SemaphoreType.DMA((NB,)),
            pltpu.SemaphoreType.DMA((NB,)),
        ]
    return pl.kernel(
        functools.partial(_sc_seg_sum_body, with_counts),
        out_type=out_type,
        mesh=mesh,
        scratch_types=scratch,
        compiler_params=pltpu.CompilerParams(use_tc_tiling_on_sc=False),
    )


def _t0_body(x_ref, w0_ref, b0_ref, w1_ref, b1_ref, w2_ref, b2_ref,
             out_ref, wf_ref):
    out_ref[...] = jax.nn.relu(
        jnp.dot(x_ref[...], w0_ref[...], preferred_element_type=jnp.float32)
        + b0_ref[...]
    )
    hidden = jax.nn.relu(w1_ref[...] + b1_ref[...])
    wf_ref[...] = (
        jnp.dot(hidden, w2_ref[...], preferred_element_type=jnp.float32)
        + b2_ref[...]
    )


_t0_call = pl.pallas_call(
    _t0_body,
    out_shape=[
        jax.ShapeDtypeStruct((N, D), jnp.float32),
        jax.ShapeDtypeStruct((1, D * D), jnp.float32),
    ],
)


def _t1_body(acc_ref, cnt_ref, h_ref, wf_ref, cb_ref,
             wih_ref, whh_ref, bih_ref, bhh_ref, out_ref):
    seg = acc_ref[0, :N, :] + acc_ref[1, :N, :]
    cnt = cnt_ref[0, :N, :] + cnt_ref[1, :N, :]
    mean = seg / jnp.maximum(cnt, 1.0)
    m = jax.nn.relu(
        jnp.dot(mean, wf_ref[...], preferred_element_type=jnp.float32)
        + cb_ref[...]
    )
    h = h_ref[...]
    gi = jnp.dot(m, wih_ref[...], preferred_element_type=jnp.float32) + bih_ref[...]
    gh = jnp.dot(h, whh_ref[...], preferred_element_type=jnp.float32) + bhh_ref[...]
    r = jax.nn.sigmoid(gi[:, :D] + gh[:, :D])
    z = jax.nn.sigmoid(gi[:, D:2 * D] + gh[:, D:2 * D])
    n = jnp.tanh(gi[:, 2 * D:] + r * gh[:, 2 * D:])
    out_ref[...] = (1.0 - z) * n + z * h


_t1_call = pl.pallas_call(
    _t1_body,
    out_shape=jax.ShapeDtypeStruct((N, D), jnp.float32),
)


def _t2_body(out_node_ref, batch_ref, wih_ref, whh_ref, bih_ref, bhh_ref,
             q_ref):
    out = out_node_ref[...]
    bidx = batch_ref[...]
    cols = lax.broadcasted_iota(jnp.int32, (N, B), 1)
    oh = (bidx == cols).astype(jnp.float32)
    q_star = jnp.zeros((B, 2 * D), jnp.float32)
    hs = jnp.zeros((B, D), jnp.float32)
    cs = jnp.zeros((B, D), jnp.float32)
    for _ in range(3):
        gates = (
            jnp.dot(q_star, wih_ref[...], preferred_element_type=jnp.float32)
            + bih_ref[...]
            + jnp.dot(hs, whh_ref[...], preferred_element_type=jnp.float32)
            + bhh_ref[...]
        )
        ig = jax.nn.sigmoid(gates[:, :D])
        fg = jax.nn.sigmoid(gates[:, D:2 * D])
        gg = jnp.tanh(gates[:, 2 * D:3 * D])
        og = jax.nn.sigmoid(gates[:, 3 * D:])
        cs = fg * cs + ig * gg
        hs = og * jnp.tanh(cs)
        qb = jnp.dot(oh, hs, preferred_element_type=jnp.float32)
        e = jnp.sum(out * qb, axis=1, keepdims=True)
        e_masked = jnp.where(oh > 0.0, e, -1e30)
        e_max = jnp.max(e_masked, axis=0, keepdims=True)
        e_max_n = jnp.dot(oh, e_max.T, preferred_element_type=jnp.float32)
        a_un = jnp.exp(e - e_max_n)
        denom = lax.dot_general(
            oh, a_un, (((0,), (0,)), ((), ())),
            preferred_element_type=jnp.float32,
        )
        den_n = jnp.dot(oh, denom, preferred_element_type=jnp.float32)
        a = a_un / (den_n + 1e-16)
        r = lax.dot_general(
            oh, a * out, (((0,), (0,)), ((), ())),
            preferred_element_type=jnp.float32,
        )
        q_star = jnp.concatenate([hs, r], axis=1)
    q_ref[...] = q_star


_t2_call = pl.pallas_call(
    _t2_body,
    out_shape=jax.ShapeDtypeStruct((B, 2 * D), jnp.float32),
)


def kernel(x, edge_index, batch, W0, b0, W1, b1, W2, b2, conv_b,
           gru_Wih, gru_Whh, gru_bih, gru_bhh,
           lstm_Wih, lstm_Whh, lstm_bih, lstm_bhh):
    src = edge_index[0]
    dst = edge_index[1]
    npad = EPAD - E
    src_p = jnp.concatenate(
        [src, jnp.zeros((npad,), jnp.int32)]).reshape(NW, NCHUNK, CHUNK)
    dst_p = jnp.concatenate(
        [dst, jnp.full((npad,), NPAD - 1, jnp.int32)]).reshape(NW, NCHUNK, CHUNK)

    out0, wf_flat = _t0_call(
        x, W0, b0.reshape(1, D), W1, b1.reshape(1, D), W2, b2.reshape(1, D * D)
    )
    wfix = wf_flat.reshape(D, D)
    cb = conv_b.reshape(1, D)
    wihT, bihT = gru_Wih.T, gru_bih.reshape(1, 3 * D)
    whhT, bhhT = gru_Whh.T, gru_bhh.reshape(1, 3 * D)

    h = out0
    cnt2 = None
    for layer in range(3):
        if layer == 0:
            acc2, cnt2 = _make_sc_seg_sum(True)(h, src_p, dst_p)
        else:
            (acc2,) = _make_sc_seg_sum(False)(h, src_p, dst_p)
        h = _t1_call(acc2, cnt2, h, wfix, cb, wihT, whhT, bihT, bhhT)

    q_star = _t2_call(
        h, batch.reshape(N, 1), lstm_Wih.T, lstm_Whh.T,
        lstm_bih.reshape(1, 4 * D), lstm_bhh.reshape(1, 4 * D),
    )
    return (q_star, h)


# speedup vs baseline: 26.4692x; 1.3635x over previous
import functools

import jax
import jax.numpy as jnp
from jax import lax
from jax.experimental import pallas as pl
from jax.experimental.pallas import tpu as pltpu
from jax.experimental.pallas import tpu_sc as plsc

N, E, F, D, B = 10000, 320000, 128, 16, 64

NC, NS = 2, 16
NW = NC * NS

CHUNK = 128
K = 4
NB = 2 * K
NCHUNK = -(-E // (NW * CHUNK * NB)) * NB
PER_W = NCHUNK * CHUNK
EPAD = NW * PER_W
NBLK = NCHUNK // NB
RPT = -(-N // NS) // 8 * 8 + 8
NPAD = RPT * NS


def _sc_seg_sum_body(with_counts, table, srcp, dstp, *refs):
    if with_counts:
        (out_acc, out_cnt, sidx2, didx2, rows, ones_v, zbuf,
         tbl_sh, acc_sh, cnt_sh, gsem, ssem, csem) = refs
    else:
        out_acc, sidx2, didx2, rows, zbuf, tbl_sh, acc_sh, gsem, ssem = refs
        out_cnt = ones_v = cnt_sh = csem = None

    c = lax.axis_index("c")
    s = lax.axis_index("s")
    w = s * NC + c

    def _zero_row(i, _):
        zbuf[i, :] = jnp.zeros((D,), jnp.float32)
        return 0

    lax.fori_loop(0, RPT, _zero_row, 0)
    pltpu.sync_copy(zbuf, acc_sh.at[pl.ds(s * RPT, RPT)])
    if with_counts:
        def _one_row(i, _):
            ones_v[i, :] = jnp.ones((D,), jnp.float32)
            return 0

        lax.fori_loop(0, CHUNK, _one_row, 0)
        pltpu.sync_copy(zbuf, cnt_sh.at[pl.ds(s * RPT, RPT)])

    pltpu.sync_copy(srcp.at[w], sidx2)
    pltpu.sync_copy(dstp.at[w], didx2)
    pltpu.sync_copy(table.at[pl.ds(s * RPT, RPT)],
                    tbl_sh.at[pl.ds(s * RPT, RPT)])
    plsc.subcore_barrier()

    def g_issue(j, b):
        pltpu.async_copy(tbl_sh.at[sidx2.at[j]], rows.at[b], gsem.at[b])

    def g_wait(j, b):
        pltpu.make_async_copy(
            tbl_sh.at[sidx2.at[j]], rows.at[b], gsem.at[b]
        ).wait()

    def s_issue(j, b):
        pltpu.async_copy(rows.at[b], acc_sh.at[didx2.at[j]], ssem.at[b],
                         add=True)
        if with_counts:
            pltpu.async_copy(ones_v, cnt_sh.at[didx2.at[j]], csem.at[b],
                             add=True)

    def s_wait(j, b):
        pltpu.make_async_copy(
            rows.at[b], acc_sh.at[didx2.at[j]], ssem.at[b]
        ).wait()
        if with_counts:
            pltpu.make_async_copy(
                ones_v, cnt_sh.at[didx2.at[j]], csem.at[b]
            ).wait()

    for b in range(NB):
        g_issue(b, b)

    def _blk(t, _):
        j0 = t * NB
        for b in range(NB):
            j = j0 + b
            g_wait(j, b)
            s_issue(j, b)
            br = (b - K) % NB
            jr = j0 + br + (NB if b >= K else 0)
            @pl.when(jnp.logical_and(jr >= NB, jr < NCHUNK))
            def _():
                s_wait(jr - NB, br)
                g_issue(jr, br)
        return 0

    lax.fori_loop(0, NBLK, _blk, 0)
    for b in range(NB):
        s_wait((NBLK - 1) * NB + b, b)
    plsc.subcore_barrier()

    pltpu.sync_copy(
        acc_sh.at[pl.ds(s * RPT, RPT)], out_acc.at[c, pl.ds(s * RPT, RPT)]
    )
    if with_counts:
        pltpu.sync_copy(
            cnt_sh.at[pl.ds(s * RPT, RPT)], out_cnt.at[c, pl.ds(s * RPT, RPT)]
        )


@functools.lru_cache(maxsize=None)
def _make_sc_seg_sum(with_counts):
    mesh = plsc.VectorSubcoreMesh(
        core_axis_name="c", subcore_axis_name="s", num_cores=NC, num_subcores=NS
    )
    out_type = [jax.ShapeDtypeStruct((NC, NPAD, D), jnp.float32)]
    if with_counts:
        out_type.append(jax.ShapeDtypeStruct((NC, NPAD, D), jnp.float32))
        scratch = [
            pltpu.VMEM((NCHUNK, CHUNK), jnp.int32),
            pltpu.VMEM((NCHUNK, CHUNK), jnp.int32),
            pltpu.VMEM((NB, CHUNK, D), jnp.float32),
            pltpu.VMEM((CHUNK, D), jnp.float32),
            pltpu.VMEM((RPT, D), jnp.float32),
            pltpu.VMEM_SHARED((NPAD, D), jnp.float32),
            pltpu.VMEM_SHARED((NPAD, D), jnp.float32),
            pltpu.VMEM_SHARED((NPAD, D), jnp.float32),
            pltpu.SemaphoreType.DMA((NB,)),
            pltpu.SemaphoreType.DMA((NB,)),
            pltpu.SemaphoreType.DMA((NB,)),
        ]
    else:
        scratch = [
            pltpu.VMEM((NCHUNK, CHUNK), jnp.int32),
            pltpu.VMEM((NCHUNK, CHUNK), jnp.int32),
            pltpu.VMEM((NB, CHUNK, D), jnp.float32),
            pltpu.VMEM((RPT, D), jnp.float32),
            pltpu.VMEM_SHARED((NPAD, D), jnp.float32),
            pltpu.VMEM_SHARED((NPAD, D), jnp.float32),
            pltpu.SemaphoreType.DMA((NB,)),
            pltpu.SemaphoreType.DMA((NB,)),
        ]
    return pl.kernel(
        functools.partial(_sc_seg_sum_body, with_counts),
        out_type=out_type,
        mesh=mesh,
        scratch_types=scratch,
        compiler_params=pltpu.CompilerParams(use_tc_tiling_on_sc=False),
    )


def _t0_body(x_ref, w0_ref, b0_ref, w1_ref, b1_ref, w2_ref, b2_ref,
             out_ref, wf_ref):
    out_ref[:N, :] = jax.nn.relu(
        jnp.dot(x_ref[...], w0_ref[...], preferred_element_type=jnp.float32)
        + b0_ref[...]
    )
    out_ref[N:, :] = jnp.zeros((NPAD - N, D), jnp.float32)
    hidden = jax.nn.relu(w1_ref[...] + b1_ref[...])
    wf_ref[...] = (
        jnp.dot(hidden, w2_ref[...], preferred_element_type=jnp.float32)
        + b2_ref[...]
    )


_t0_call = pl.pallas_call(
    _t0_body,
    out_shape=[
        jax.ShapeDtypeStruct((NPAD, D), jnp.float32),
        jax.ShapeDtypeStruct((1, D * D), jnp.float32),
    ],
)


def _t1_body(acc_ref, cnt_ref, h_ref, wf_ref, cb_ref,
             wih_ref, whh_ref, bih_ref, bhh_ref, out_ref):
    seg = acc_ref[0, :N, :] + acc_ref[1, :N, :]
    cnt = cnt_ref[0, :N, :] + cnt_ref[1, :N, :]
    mean = seg / jnp.maximum(cnt, 1.0)
    m = jax.nn.relu(
        jnp.dot(mean, wf_ref[...], preferred_element_type=jnp.float32)
        + cb_ref[...]
    )
    h = h_ref[:N, :]
    gi = jnp.dot(m, wih_ref[...], preferred_element_type=jnp.float32) + bih_ref[...]
    gh = jnp.dot(h, whh_ref[...], preferred_element_type=jnp.float32) + bhh_ref[...]
    r = jax.nn.sigmoid(gi[:, :D] + gh[:, :D])
    z = jax.nn.sigmoid(gi[:, D:2 * D] + gh[:, D:2 * D])
    n = jnp.tanh(gi[:, 2 * D:] + r * gh[:, 2 * D:])
    out_ref[:N, :] = (1.0 - z) * n + z * h
    out_ref[N:, :] = jnp.zeros((NPAD - N, D), jnp.float32)


_t1_call = pl.pallas_call(
    _t1_body,
    out_shape=jax.ShapeDtypeStruct((NPAD, D), jnp.float32),
)


def _t2_body(out_node_ref, batch_ref, wih_ref, whh_ref, bih_ref, bhh_ref,
             q_ref):
    out = out_node_ref[:N, :]
    bidx = batch_ref[...]
    cols = lax.broadcasted_iota(jnp.int32, (N, B), 1)
    oh = (bidx == cols).astype(jnp.float32)
    q_star = jnp.zeros((B, 2 * D), jnp.float32)
    hs = jnp.zeros((B, D), jnp.float32)
    cs = jnp.zeros((B, D), jnp.float32)
    for _ in range(3):
        gates = (
            jnp.dot(q_star, wih_ref[...], preferred_element_type=jnp.float32)
            + bih_ref[...]
            + jnp.dot(hs, whh_ref[...], preferred_element_type=jnp.float32)
            + bhh_ref[...]
        )
        ig = jax.nn.sigmoid(gates[:, :D])
        fg = jax.nn.sigmoid(gates[:, D:2 * D])
        gg = jnp.tanh(gates[:, 2 * D:3 * D])
        og = jax.nn.sigmoid(gates[:, 3 * D:])
        cs = fg * cs + ig * gg
        hs = og * jnp.tanh(cs)
        qb = jnp.dot(oh, hs, preferred_element_type=jnp.float32)
        e = jnp.sum(out * qb, axis=1, keepdims=True)
        e_masked = jnp.where(oh > 0.0, e, -1e30)
        e_max = jnp.max(e_masked, axis=0, keepdims=True)
        e_max_n = jnp.dot(oh, e_max.T, preferred_element_type=jnp.float32)
        a_un = jnp.exp(e - e_max_n)
        denom = lax.dot_general(
            oh, a_un, (((0,), (0,)), ((), ())),
            preferred_element_type=jnp.float32,
        )
        den_n = jnp.dot(oh, denom, preferred_element_type=jnp.float32)
        a = a_un / (den_n + 1e-16)
        r = lax.dot_general(
            oh, a * out, (((0,), (0,)), ((), ())),
            preferred_element_type=jnp.float32,
        )
        q_star = jnp.concatenate([hs, r], axis=1)
    q_ref[...] = q_star


_t2_call = pl.pallas_call(
    _t2_body,
    out_shape=jax.ShapeDtypeStruct((B, 2 * D), jnp.float32),
)


def kernel(x, edge_index, batch, W0, b0, W1, b1, W2, b2, conv_b,
           gru_Wih, gru_Whh, gru_bih, gru_bhh,
           lstm_Wih, lstm_Whh, lstm_bih, lstm_bhh):
    src = edge_index[0]
    dst = edge_index[1]
    npad = EPAD - E
    src_p = jnp.concatenate(
        [src, jnp.zeros((npad,), jnp.int32)]).reshape(NW, NCHUNK, CHUNK)
    dst_p = jnp.concatenate(
        [dst, jnp.full((npad,), NPAD - 1, jnp.int32)]).reshape(NW, NCHUNK, CHUNK)

    out0, wf_flat = _t0_call(
        x, W0, b0.reshape(1, D), W1, b1.reshape(1, D), W2, b2.reshape(1, D * D)
    )
    wfix = wf_flat.reshape(D, D)
    cb = conv_b.reshape(1, D)
    wihT, bihT = gru_Wih.T, gru_bih.reshape(1, 3 * D)
    whhT, bhhT = gru_Whh.T, gru_bhh.reshape(1, 3 * D)

    h = out0
    cnt2 = None
    for layer in range(3):
        if layer == 0:
            acc2, cnt2 = _make_sc_seg_sum(True)(h, src_p, dst_p)
        else:
            (acc2,) = _make_sc_seg_sum(False)(h, src_p, dst_p)
        h = _t1_call(acc2, cnt2, h, wfix, cb, wihT, whhT, bihT, bhhT)

    q_star = _t2_call(
        h, batch.reshape(N, 1), lstm_Wih.T, lstm_Whh.T,
        lstm_bih.reshape(1, 4 * D), lstm_bhh.reshape(1, 4 * D),
    )
    return (q_star, h[:N, :])
